# Initial kernel scaffold; baseline (speedup 1.0000x reference)
import jax
import jax.numpy as jnp
from jax.experimental import pallas as pl


def kernel(word_embs, input_ids, class_ids, mask, co_edge_index, co_edge_weight, re_edge_index, re_edge_weight, pnc_graph, co_W, co_b, re_W, re_b, pnc_W, pnc_b, zc_W, zc_b, zcp_W, zcp_b):
    raise NotImplementedError("write your pallas kernel here")


# speedup vs baseline: 1.2215x; 1.2215x over previous
import functools

import jax
import jax.numpy as jnp
from jax import lax
from jax.experimental import pallas as pl
from jax.experimental.pallas import tpu as pltpu

N = 10000
E = 320000
D = 128
B = 4
L = 512
C = 1000
T = 12


def _assemble_body(x_ref, w_ref, b_ref, m_ref, o_ref):
    x = x_ref[0, 0]
    w = w_ref[0, 0]
    y = lax.dot_general(x, w, (((1,), (1,)), ((), ())),
                        preferred_element_type=jnp.float32)
    y = (y + b_ref[0, 0, 0]) * m_ref[0, 0]
    o_ref[0, 0, 0, 0:1, :] = jnp.zeros((1, D), jnp.float32)
    o_ref[0, 0, 0, 1:, :] = y


def _assemble(embs_all, w_all, b_all, m_all):
    grid = (3, T, B)
    return pl.pallas_call(
        _assemble_body,
        grid=grid,
        in_specs=[
            pl.BlockSpec((1, 1, L, D), lambda i, t, b: (i, b, 0, 0)),
            pl.BlockSpec((1, 1, D, D), lambda i, t, b: (i, t, 0, 0)),
            pl.BlockSpec((1, 1, 1, D), lambda i, t, b: (i, t, 0, 0)),
            pl.BlockSpec((1, 1, L, 1), lambda i, t, b: (i, b, 0, 0)),
        ],
        out_specs=pl.BlockSpec((1, 1, 1, L + 1, D),
                               lambda i, t, b: (i, t, b, 0, 0)),
        out_shape=jax.ShapeDtypeStruct((3, T, B, L + 1, D), jnp.float32),
    )(embs_all, w_all, b_all, m_all)


def _gcn_forward(inputs_idx, x, src, dst, ew, Ws, bs):
    deg = jnp.zeros((N,), jnp.float32).at[dst].add(ew) + 1.0
    dinv = deg ** -0.5
    norm = dinv[src] * ew * dinv[dst]
    self_w = dinv * dinv
    acc = jnp.zeros((B * L, D), jnp.float32)
    for i in range(3):
        h = x @ Ws[i].T
        agg = jnp.zeros((N, D), jnp.float32).at[dst].add(norm[:, None] * h[src])
        agg = agg + self_w[:, None] * h + bs[i]
        x = jax.nn.relu(agg)
        x_pad = jnp.concatenate([x, jnp.zeros((1, D), jnp.float32)], axis=0)
        acc = acc + x_pad[inputs_idx.reshape(-1)]
    return (acc * (1.0 / 3.0)).reshape(B, L, D)


def kernel(word_embs, input_ids, class_ids, mask, co_edge_index,
           co_edge_weight, re_edge_index, re_edge_weight, pnc_graph,
           co_W, co_b, re_W, re_b, pnc_W, pnc_b, zc_W, zc_b, zcp_W, zcp_b):
    co_embs = _gcn_forward(input_ids, word_embs, co_edge_index[0],
                           co_edge_index[1], co_edge_weight, co_W, co_b)
    re_embs = _gcn_forward(input_ids, word_embs, re_edge_index[0],
                           re_edge_index[1], re_edge_weight, re_W, re_b)

    codebook_conv = word_embs @ pnc_W.T + pnc_b
    classmat = pnc_graph @ codebook_conv
    pn = classmat[class_ids.reshape(-1)].reshape(B, L, D)

    co_re_mask = (~mask).astype(jnp.float32)
    embs_all = jnp.stack([co_embs, re_embs, pn], axis=0)
    w_all = jnp.stack([zc_W, zc_W, zcp_W], axis=0)
    b_all = jnp.stack([zc_b, zc_b, zcp_b], axis=0)[:, :, None, :]
    m_all = jnp.stack([co_re_mask, co_re_mask,
                       jnp.ones_like(co_re_mask)], axis=0)

    return _assemble(embs_all, w_all, b_all, m_all)


# speedup vs baseline: 1.6808x; 1.3760x over previous
import functools

import jax
import jax.numpy as jnp
from jax import lax
from jax.experimental import pallas as pl
from jax.experimental.pallas import tpu as pltpu
from jax.experimental.pallas import tpu_sc as plsc

N = 10000
E = 320000
D = 128
B = 4
L = 512
C = 1000
T = 12

NC = 2
NS = 16
LN = 16


_EW = E // NS
_KC = 80
_NSTG = 25
_NBLK = _EW // (_KC * _NSTG)
_ZR = 128


def _sc_aggregate(h_all, src3d, dst3d, norm3d):
    mesh = plsc.VectorSubcoreMesh(core_axis_name="c", subcore_axis_name="s",
                                  num_cores=NC, num_subcores=NS)

    @functools.partial(
        pl.kernel, mesh=mesh,
        out_type=jax.ShapeDtypeStruct((NC, N, D), jnp.float32),
        scratch_types=[
            pltpu.VMEM((_NSTG, _KC), jnp.int32),
            pltpu.VMEM((_NSTG, _KC), jnp.int32),
            pltpu.VMEM((_NSTG, _KC), jnp.float32),
            pltpu.VMEM((_KC, D), jnp.float32),
            pltpu.VMEM((_ZR, D), jnp.float32),
            pltpu.VMEM_SHARED((N, D), jnp.float32),
            pltpu.SemaphoreType.DMA,
        ],
    )
    def agg(h_hbm, src_hbm, dst_hbm, norm_hbm, out_hbm,
            src_v, dst_v, norm_v, rows_v, zero_v, acc_sh, sem):
        c = lax.axis_index("c")
        s = lax.axis_index("s")

        zeros16 = jnp.zeros((LN,), jnp.float32)

        def zrow(i, _):
            for m in range(D // LN):
                zero_v[i, pl.ds(m * LN, LN)] = zeros16
            return 0
        lax.fori_loop(0, _ZR, zrow, 0)
        @pl.when(s < NS - 1)
        def _():
            for k in range(5):
                pltpu.sync_copy(zero_v,
                                acc_sh.at[pl.ds(s * 640 + k * _ZR, _ZR)])

        @pl.when(s == NS - 1)
        def _():
            for k in range(3):
                pltpu.sync_copy(zero_v,
                                acc_sh.at[pl.ds(9600 + k * _ZR, _ZR)])
            pltpu.sync_copy(zero_v.at[pl.ds(0, 16)],
                            acc_sh.at[pl.ds(9984, 16)])
        plsc.subcore_barrier()

        blk0 = c * (NS * _NBLK) + s * _NBLK

        def blk(b, _):
            r = blk0 + b
            pltpu.sync_copy(src_hbm.at[r], src_v)
            pltpu.sync_copy(dst_hbm.at[r], dst_v)
            pltpu.sync_copy(norm_hbm.at[r], norm_v)

            def chunk(j, _):
                pltpu.async_copy(h_hbm.at[src_v.at[j]], rows_v, sem).wait()

                def grp(g, _):
                    base = g * LN
                    norm16 = norm_v[j, pl.ds(base, LN)]
                    for k in range(LN):
                        sc = norm16[k]
                        for m in range(D // LN):
                            sl = pl.ds(m * LN, LN)
                            rows_v[base + k, sl] = rows_v[base + k, sl] * sc
                    return 0
                lax.fori_loop(0, _KC // LN, grp, 0)
                pltpu.sync_copy(rows_v, acc_sh.at[dst_v.at[j]], add=True)
                return 0
            lax.fori_loop(0, _NSTG, chunk, 0)
            return 0
        lax.fori_loop(0, _NBLK, blk, 0)

        plsc.subcore_barrier()
        @pl.when(s < NS - 1)
        def _():
            for k in range(5):
                sl = pl.ds(s * 640 + k * _ZR, _ZR)
                pltpu.sync_copy(acc_sh.at[sl], out_hbm.at[c].at[sl])

        @pl.when(s == NS - 1)
        def _():
            for k in range(3):
                sl = pl.ds(9600 + k * _ZR, _ZR)
                pltpu.sync_copy(acc_sh.at[sl], out_hbm.at[c].at[sl])
            sl = pl.ds(9984, 16)
            pltpu.sync_copy(acc_sh.at[sl], out_hbm.at[c].at[sl])

    return agg(h_all, src3d, dst3d, norm3d)


def _assemble_body(x_ref, w_ref, b_ref, m_ref, o_ref):
    x = x_ref[0, 0]
    w = w_ref[0, 0]
    y = lax.dot_general(x, w, (((1,), (1,)), ((), ())),
                        preferred_element_type=jnp.float32)
    y = (y + b_ref[0, 0, 0]) * m_ref[0, 0]
    o_ref[0, 0, 0, 0:1, :] = jnp.zeros((1, D), jnp.float32)
    o_ref[0, 0, 0, 1:, :] = y


def _assemble(embs_all, w_all, b_all, m_all):
    grid = (3, T, B)
    return pl.pallas_call(
        _assemble_body,
        grid=grid,
        in_specs=[
            pl.BlockSpec((1, 1, L, D), lambda i, t, b: (i, b, 0, 0)),
            pl.BlockSpec((1, 1, D, D), lambda i, t, b: (i, t, 0, 0)),
            pl.BlockSpec((1, 1, 1, D), lambda i, t, b: (i, t, 0, 0)),
            pl.BlockSpec((1, 1, L, 1), lambda i, t, b: (i, b, 0, 0)),
        ],
        out_specs=pl.BlockSpec((1, 1, 1, L + 1, D),
                               lambda i, t, b: (i, t, b, 0, 0)),
        out_shape=jax.ShapeDtypeStruct((3, T, B, L + 1, D), jnp.float32),
    )(embs_all, w_all, b_all, m_all)


def _edge_prep(edge_index, ew):
    src, dst = edge_index[0], edge_index[1]
    deg = jnp.zeros((N,), jnp.float32).at[dst].add(ew) + 1.0
    dinv = deg ** -0.5
    norm = dinv[src] * ew * dinv[dst]
    return src, dst, norm, dinv


def _gcn_both(input_ids, word_embs, co_edge_index, co_edge_weight,
              re_edge_index, re_edge_weight, co_W, co_b, re_W, re_b):
    co_src, co_dst, co_norm, co_dinv = _edge_prep(co_edge_index,
                                                  co_edge_weight)
    re_src, re_dst, re_norm, re_dinv = _edge_prep(re_edge_index,
                                                  re_edge_weight)
    src3d = jnp.concatenate([co_src, re_src + N]).reshape(-1, _NSTG, _KC)
    dst3d = jnp.concatenate([co_dst, re_dst]).reshape(-1, _NSTG, _KC)
    norm3d = jnp.concatenate([co_norm, re_norm]).reshape(-1, _NSTG, _KC)
    selfw = jnp.stack([co_dinv, re_dinv])[:, :, None] ** 2
    W2 = jnp.stack([co_W, re_W])
    b2 = jnp.stack([co_b, re_b])

    idx_flat = input_ids.reshape(-1)
    x = jnp.broadcast_to(word_embs[None], (2, N, D))
    acc = jnp.zeros((2, B * L, D), jnp.float32)
    for layer in range(3):
        h = jnp.einsum('gnd,gkd->gnk', x, W2[:, layer],
                       preferred_element_type=jnp.float32)
        agg = _sc_aggregate(h.reshape(2 * N, D), src3d, dst3d, norm3d)
        x = jax.nn.relu(agg + selfw * h + b2[:, layer][:, None, :])
        xpad = jnp.concatenate(
            [x, jnp.zeros((2, 1, D), jnp.float32)], axis=1)
        acc = acc + xpad[:, idx_flat]
    acc = acc * (1.0 / 3.0)
    return acc[0].reshape(B, L, D), acc[1].reshape(B, L, D)


def kernel(word_embs, input_ids, class_ids, mask, co_edge_index,
           co_edge_weight, re_edge_index, re_edge_weight, pnc_graph,
           co_W, co_b, re_W, re_b, pnc_W, pnc_b, zc_W, zc_b, zcp_W, zcp_b):
    co_embs, re_embs = _gcn_both(input_ids, word_embs, co_edge_index,
                                 co_edge_weight, re_edge_index,
                                 re_edge_weight, co_W, co_b, re_W, re_b)


    codebook_conv = word_embs @ pnc_W.T + pnc_b
    classmat = pnc_graph @ codebook_conv
    pn = classmat[class_ids.reshape(-1)].reshape(B, L, D)

    co_re_mask = (~mask).astype(jnp.float32)
    embs_all = jnp.stack([co_embs, re_embs, pn], axis=0)
    w_all = jnp.stack([zc_W, zc_W, zcp_W], axis=0)
    b_all = jnp.stack([zc_b, zc_b, zcp_b], axis=0)[:, :, None, :]
    m_all = jnp.stack([co_re_mask, co_re_mask,
                       jnp.ones_like(co_re_mask)], axis=0)

    return _assemble(embs_all, w_all, b_all, m_all)


# speedup vs baseline: 9.5912x; 5.7063x over previous
import functools

import jax
import jax.numpy as jnp
from jax import lax
from jax.experimental import pallas as pl
from jax.experimental.pallas import tpu as pltpu
from jax.experimental.pallas import tpu_sc as plsc

N = 10000
E = 320000
D = 128
B = 4
L = 512
C = 1000
T = 12

NC = 2
NS = 16
LN = 16


_EW = E // NS
_KC = 80
_NSTG = 25
_NBLK = _EW // (_KC * _NSTG)
_ZR = 128


def _sc_aggregate(h_all, src3d, dst3d, norm3d):
    mesh = plsc.VectorSubcoreMesh(core_axis_name="c", subcore_axis_name="s",
                                  num_cores=NC, num_subcores=NS)

    @functools.partial(
        pl.kernel, mesh=mesh,
        out_type=jax.ShapeDtypeStruct((NC, N, D), jnp.float32),
        scratch_types=[
            pltpu.VMEM((_NSTG, _KC), jnp.int32),
            pltpu.VMEM((_NSTG, _KC), jnp.int32),
            pltpu.VMEM((_NSTG, _KC), jnp.float32),
            pltpu.VMEM((_KC, D), jnp.float32),
            pltpu.VMEM((_ZR, D), jnp.float32),
            pltpu.VMEM_SHARED((N, D), jnp.float32),
            pltpu.SemaphoreType.DMA,
        ],
    )
    def agg(h_hbm, src_hbm, dst_hbm, norm_hbm, out_hbm,
            src_v, dst_v, norm_v, rows_v, zero_v, acc_sh, sem):
        c = lax.axis_index("c")
        s = lax.axis_index("s")

        zeros16 = jnp.zeros((LN,), jnp.float32)

        def zrow(i, _):
            for m in range(D // LN):
                zero_v[i, pl.ds(m * LN, LN)] = zeros16
            return 0
        lax.fori_loop(0, _ZR, zrow, 0)
        @pl.when(s < NS - 1)
        def _():
            for k in range(5):
                pltpu.sync_copy(zero_v,
                                acc_sh.at[pl.ds(s * 640 + k * _ZR, _ZR)])

        @pl.when(s == NS - 1)
        def _():
            for k in range(3):
                pltpu.sync_copy(zero_v,
                                acc_sh.at[pl.ds(9600 + k * _ZR, _ZR)])
            pltpu.sync_copy(zero_v.at[pl.ds(0, 16)],
                            acc_sh.at[pl.ds(9984, 16)])
        plsc.subcore_barrier()

        blk0 = c * (NS * _NBLK) + s * _NBLK

        def blk(b, _):
            r = blk0 + b
            pltpu.sync_copy(src_hbm.at[r], src_v)
            pltpu.sync_copy(dst_hbm.at[r], dst_v)
            pltpu.sync_copy(norm_hbm.at[r], norm_v)

            def chunk(j, _):
                pltpu.async_copy(h_hbm.at[src_v.at[j]], rows_v, sem).wait()

                def grp(g, _):
                    base = g * LN
                    norm16 = norm_v[j, pl.ds(base, LN)]
                    for k in range(LN):
                        sc = norm16[k]
                        for m in range(D // LN):
                            sl = pl.ds(m * LN, LN)
                            rows_v[base + k, sl] = rows_v[base + k, sl] * sc
                    return 0
                lax.fori_loop(0, _KC // LN, grp, 0)
                pltpu.sync_copy(rows_v, acc_sh.at[dst_v.at[j]], add=True)
                return 0
            lax.fori_loop(0, _NSTG, chunk, 0)
            return 0
        lax.fori_loop(0, _NBLK, blk, 0)

        plsc.subcore_barrier()
        @pl.when(s < NS - 1)
        def _():
            for k in range(5):
                sl = pl.ds(s * 640 + k * _ZR, _ZR)
                pltpu.sync_copy(acc_sh.at[sl], out_hbm.at[c].at[sl])

        @pl.when(s == NS - 1)
        def _():
            for k in range(3):
                sl = pl.ds(9600 + k * _ZR, _ZR)
                pltpu.sync_copy(acc_sh.at[sl], out_hbm.at[c].at[sl])
            sl = pl.ds(9984, 16)
            pltpu.sync_copy(acc_sh.at[sl], out_hbm.at[c].at[sl])

    return agg(h_all, src3d, dst3d, norm3d)


def _sc_deg(dst3d, ew3d):
    mesh = plsc.VectorSubcoreMesh(core_axis_name="c", subcore_axis_name="s",
                                  num_cores=NC, num_subcores=NS)

    @functools.partial(
        pl.kernel, mesh=mesh,
        out_type=jax.ShapeDtypeStruct((NC, N, D), jnp.float32),
        scratch_types=[
            pltpu.VMEM((_NSTG, _KC), jnp.int32),
            pltpu.VMEM((_NSTG, _KC), jnp.float32),
            pltpu.VMEM((_KC, D), jnp.float32),
            pltpu.VMEM((_ZR, D), jnp.float32),
            pltpu.VMEM_SHARED((N, D), jnp.float32),
        ],
    )
    def deg(dst_hbm, ew_hbm, deg_hbm, dst_v, ew_v, upd_v, zero_v, deg_sh):
        c = lax.axis_index("c")
        s = lax.axis_index("s")
        zeros16 = jnp.zeros((LN,), jnp.float32)

        def zup(i, _):
            for m in range(D // LN):
                upd_v[i, pl.ds(m * LN, LN)] = zeros16
            return 0
        lax.fori_loop(0, _KC, zup, 0)

        def zrow(i, _):
            for m in range(D // LN):
                zero_v[i, pl.ds(m * LN, LN)] = zeros16
            return 0
        lax.fori_loop(0, _ZR, zrow, 0)

        @pl.when(s < NS - 1)
        def _():
            for k in range(5):
                pltpu.sync_copy(zero_v,
                                deg_sh.at[pl.ds(s * 640 + k * _ZR, _ZR)])

        @pl.when(s == NS - 1)
        def _():
            for k in range(3):
                pltpu.sync_copy(zero_v,
                                deg_sh.at[pl.ds(9600 + k * _ZR, _ZR)])
            pltpu.sync_copy(zero_v.at[pl.ds(0, 16)],
                            deg_sh.at[pl.ds(9984, 16)])
        plsc.subcore_barrier()

        blk0 = c * (NS * _NBLK) + s * _NBLK

        def blk1(b, _):
            r = blk0 + b
            pltpu.sync_copy(dst_hbm.at[r], dst_v)
            pltpu.sync_copy(ew_hbm.at[r], ew_v)

            def chunk(j, _):
                for g in range(_KC // LN):
                    ew16 = ew_v[j, pl.ds(g * LN, LN)]
                    for k in range(LN):
                        upd_v[g * LN + k, pl.ds(0, LN)] = jnp.full(
                            (LN,), ew16[k], jnp.float32)
                pltpu.sync_copy(upd_v, deg_sh.at[dst_v.at[j]], add=True)
                return 0
            lax.fori_loop(0, _NSTG, chunk, 0)
            return 0
        lax.fori_loop(0, _NBLK, blk1, 0)
        plsc.subcore_barrier()

        @pl.when(s < NS - 1)
        def _():
            for k in range(5):
                sl = pl.ds(s * 640 + k * _ZR, _ZR)
                pltpu.sync_copy(deg_sh.at[sl], deg_hbm.at[c].at[sl])

        @pl.when(s == NS - 1)
        def _():
            for k in range(3):
                sl = pl.ds(9600 + k * _ZR, _ZR)
                pltpu.sync_copy(deg_sh.at[sl], deg_hbm.at[c].at[sl])
            sl = pl.ds(9984, 16)
            pltpu.sync_copy(deg_sh.at[sl], deg_hbm.at[c].at[sl])

    return deg(dst3d, ew3d)


def _tc_dinv_body(deg_ref, c_ref):
    d = deg_ref[:, :, 0] + 1.0
    c_ref[...] = lax.rsqrt(d)


def _tc_dinv(deg16):
    return pl.pallas_call(
        _tc_dinv_body,
        out_shape=jax.ShapeDtypeStruct((NC, N), jnp.float32),
    )(deg16)


def _assemble_body(x_ref, w_ref, b_ref, m_ref, o_ref):
    x = x_ref[0, 0]
    w = w_ref[0, 0]
    y = lax.dot_general(x, w, (((1,), (1,)), ((), ())),
                        preferred_element_type=jnp.float32)
    y = (y + b_ref[0, 0, 0]) * m_ref[0, 0]
    o_ref[0, 0, 0, 0:1, :] = jnp.zeros((1, D), jnp.float32)
    o_ref[0, 0, 0, 1:, :] = y


def _assemble(embs_all, w_all, b_all, m_all):
    grid = (3, T, B)
    return pl.pallas_call(
        _assemble_body,
        grid=grid,
        in_specs=[
            pl.BlockSpec((1, 1, L, D), lambda i, t, b: (i, b, 0, 0)),
            pl.BlockSpec((1, 1, D, D), lambda i, t, b: (i, t, 0, 0)),
            pl.BlockSpec((1, 1, 1, D), lambda i, t, b: (i, t, 0, 0)),
            pl.BlockSpec((1, 1, L, 1), lambda i, t, b: (i, b, 0, 0)),
        ],
        out_specs=pl.BlockSpec((1, 1, 1, L + 1, D),
                               lambda i, t, b: (i, t, b, 0, 0)),
        out_shape=jax.ShapeDtypeStruct((3, T, B, L + 1, D), jnp.float32),
    )(embs_all, w_all, b_all, m_all)


def _gcn_both(input_ids, word_embs, co_edge_index, co_edge_weight,
              re_edge_index, re_edge_weight, co_W, co_b, re_W, re_b):
    co_src, co_dst = co_edge_index[0], co_edge_index[1]
    re_src, re_dst = re_edge_index[0], re_edge_index[1]
    srcl3d = jnp.concatenate([co_src, re_src]).reshape(-1, _NSTG, _KC)
    src3d = jnp.concatenate([co_src, re_src + N]).reshape(-1, _NSTG, _KC)
    dst3d = jnp.concatenate([co_dst, re_dst]).reshape(-1, _NSTG, _KC)
    ew3d = jnp.concatenate([co_edge_weight,
                            re_edge_weight]).reshape(-1, _NSTG, _KC)
    deg16 = _sc_deg(dst3d, ew3d)
    dinv = _tc_dinv(deg16)[:, :, None]
    selfw = dinv ** 2
    W2 = jnp.stack([co_W, re_W])
    b2 = jnp.stack([co_b, re_b])

    idx_flat = input_ids.reshape(-1)
    x = jnp.broadcast_to(word_embs[None], (2, N, D))
    acc = jnp.zeros((2, B * L, D), jnp.float32)
    for layer in range(3):
        h = jnp.einsum('gnd,gkd->gnk', x, W2[:, layer],
                       preferred_element_type=jnp.float32)
        hs = dinv * h
        agg = _sc_aggregate(hs.reshape(2 * N, D), src3d, dst3d, ew3d)
        x = jax.nn.relu(dinv * agg + selfw * h + b2[:, layer][:, None, :])
        xpad = jnp.concatenate(
            [x, jnp.zeros((2, 1, D), jnp.float32)], axis=1)
        acc = acc + xpad[:, idx_flat]
    acc = acc * (1.0 / 3.0)
    return acc[0].reshape(B, L, D), acc[1].reshape(B, L, D)


def kernel(word_embs, input_ids, class_ids, mask, co_edge_index,
           co_edge_weight, re_edge_index, re_edge_weight, pnc_graph,
           co_W, co_b, re_W, re_b, pnc_W, pnc_b, zc_W, zc_b, zcp_W, zcp_b):
    co_embs, re_embs = _gcn_both(input_ids, word_embs, co_edge_index,
                                 co_edge_weight, re_edge_index,
                                 re_edge_weight, co_W, co_b, re_W, re_b)


    codebook_conv = word_embs @ pnc_W.T + pnc_b
    classmat = pnc_graph @ codebook_conv
    pn = classmat[class_ids.reshape(-1)].reshape(B, L, D)

    co_re_mask = (~mask).astype(jnp.float32)
    embs_all = jnp.stack([co_embs, re_embs, pn], axis=0)
    w_all = jnp.stack([zc_W, zc_W, zcp_W], axis=0)
    b_all = jnp.stack([zc_b, zc_b, zcp_b], axis=0)[:, :, None, :]
    m_all = jnp.stack([co_re_mask, co_re_mask,
                       jnp.ones_like(co_re_mask)], axis=0)

    return _assemble(embs_all, w_all, b_all, m_all)
